# baseline (speedup 1.0000x reference)
import functools

import jax
import jax.numpy as jnp
from jax import lax
from jax.experimental import pallas as pl
from jax.experimental.pallas import tpu as pltpu
from jax.experimental.pallas import tpu_sc as plsc

N = 10000
E = 320000
D = 128
NC = 2
NS = 16
NW = NC * NS
EPW = E // NW
B = 128
CB = 8
NCH = 10
NB = NCH * CB
PAD = NB * B - EPW
NP = 10240
RPT = NP // NS

_mesh = plsc.VectorSubcoreMesh(core_axis_name="c", subcore_axis_name="s")


@functools.partial(
    pl.kernel,
    out_type=jax.ShapeDtypeStruct((NC, NP), jnp.float32),
    mesh=_mesh,
    scratch_types=[
        pltpu.VMEM((NB, B), jnp.int32),
        pltpu.VMEM((B,), jnp.float32),
        pltpu.VMEM((RPT,), jnp.float32),
        pltpu.VMEM_SHARED((NP,), jnp.float32),
    ],
)
def _deg_sc(dst_hbm, deg_hbm, dstv, ones_v, buf_v, deg_sh):
    c = lax.axis_index("c")
    s = lax.axis_index("s")
    wid = s * NC + c
    for i in range(B // 16):
        ones_v[pl.ds(16 * i, 16)] = jnp.ones((16,), jnp.float32)
    for i in range(RPT // 16):
        buf_v[pl.ds(16 * i, 16)] = jnp.zeros((16,), jnp.float32)
    pltpu.sync_copy(buf_v, deg_sh.at[pl.ds(s * RPT, RPT)])
    pltpu.sync_copy(dst_hbm.at[wid], dstv)
    plsc.subcore_barrier()
    for j in range(NB):
        pltpu.sync_copy(ones_v, deg_sh.at[dstv.at[j]], add=True)
    plsc.subcore_barrier()
    pltpu.sync_copy(deg_sh.at[pl.ds(s * RPT, RPT)], buf_v)
    pltpu.sync_copy(buf_v, deg_hbm.at[c, pl.ds(s * RPT, RPT)])


@functools.partial(
    pl.kernel,
    out_type=jax.ShapeDtypeStruct((NC, NP, D), jnp.float32),
    mesh=_mesh,
    scratch_types=[
        pltpu.VMEM((NB, B), jnp.int32),
        pltpu.VMEM((CB, B), jnp.int32),
        pltpu.VMEM((CB, B), jnp.int32),
        pltpu.VMEM((B, D), jnp.float32),
        pltpu.VMEM((B, D), jnp.float32),
        pltpu.VMEM_SHARED((NP, D), jnp.float32),
        pltpu.SemaphoreType.DMA,
        pltpu.SemaphoreType.DMA,
        pltpu.SemaphoreType.DMA,
        pltpu.SemaphoreType.DMA,
        pltpu.SemaphoreType.DMA,
    ],
)
def _agg_sc(y_hbm, src_hbm, dst_hbm, zrows_hbm, z_hbm,
            dstv, s0, s1, r0, r1, z_sh, gsem0, gsem1, ssem0, ssem1, isem):
    c = lax.axis_index("c")
    s = lax.axis_index("s")
    wid = s * NC + c
    svs, rows = (s0, s1), (r0, r1)
    gsems, ssems = (gsem0, gsem1), (ssem0, ssem1)
    for k in range(RPT // B):
        pltpu.sync_copy(zrows_hbm, z_sh.at[pl.ds(s * RPT + k * B, B)])
    pltpu.sync_copy(dst_hbm.at[wid], dstv)
    pltpu.sync_copy(src_hbm.at[wid, 0], s0)
    plsc.subcore_barrier()

    def gather(g, p):
        cc = g // CB
        return pltpu.async_copy(
            y_hbm.at[svs[cc % 2].at[g % CB]], rows[p], gsems[p])

    icp = [None]
    gcp = [None, None]
    gcp[0] = gather(0, 0)
    gcp[1] = gather(1, 1)
    for ch in range(NCH):
        cp = ch % 2
        if ch + 1 < NCH:
            icp[0] = pltpu.async_copy(src_hbm.at[wid, ch + 1], svs[1 - cp], isem)
        for j in range(CB):
            g = ch * CB + j
            p = g % 2
            gcp[p].wait()
            pltpu.sync_copy(rows[p], z_sh.at[dstv.at[g]], add=True)
            if g + 2 < NB:
                if j == CB - 2:
                    icp[0].wait()
                gcp[p] = gather(g + 2, p)
    plsc.subcore_barrier()
    pltpu.sync_copy(
        z_sh.at[pl.ds(s * RPT, RPT)], z_hbm.at[c, pl.ds(s * RPT, RPT)])


def _prescale_tc(dp_ref, x_ref, dinv_ref, y_ref):
    deg = dp_ref[0, :N] + dp_ref[1, :N] + 1.0
    dinv = lax.rsqrt(deg)
    dinv_ref[...] = dinv
    y_ref[...] = x_ref[...] * dinv


def _prescale(deg_parts, x):
    return pl.pallas_call(
        _prescale_tc,
        out_shape=[
            jax.ShapeDtypeStruct((N, 1), jnp.float32),
            jax.ShapeDtypeStruct((N, D), jnp.float32),
        ],
    )(deg_parts, x)


def _mid_tc(z_ref, y_ref, dinv_ref, w_ref, b_ref, y2_ref):
    dinv = dinv_ref[...]
    p = (z_ref[0, :N] + z_ref[1, :N] + y_ref[...]) * dinv
    h = jnp.maximum(
        jnp.dot(p, w_ref[...], preferred_element_type=jnp.float32) + b_ref[...],
        0.0,
    )
    y2_ref[...] = h * dinv


def _mid(z1, y1, dinv, W1, b1):
    return pl.pallas_call(
        _mid_tc,
        out_shape=jax.ShapeDtypeStruct((N, D), jnp.float32),
    )(z1, y1, dinv, W1, b1.reshape(1, D))


def _out_tc(z_ref, y_ref, dinv_ref, wm_ref, bm_ref, wl_ref, bl_ref,
            mu_ref, ls_ref):
    q = (z_ref[0, :N] + z_ref[1, :N] + y_ref[...]) * dinv_ref[...]
    mu_ref[...] = (
        jnp.dot(q, wm_ref[...], preferred_element_type=jnp.float32) + bm_ref[...]
    )
    ls_ref[...] = (
        jnp.dot(q, wl_ref[...], preferred_element_type=jnp.float32) + bl_ref[...]
    )


def _heads(z2, y2, dinv, Wmu, bmu, Wls, bls):
    do = Wmu.shape[1]
    return pl.pallas_call(
        _out_tc,
        out_shape=[
            jax.ShapeDtypeStruct((N, do), jnp.float32),
            jax.ShapeDtypeStruct((N, do), jnp.float32),
        ],
    )(z2, y2, dinv, Wmu, bmu.reshape(1, do), Wls, bls.reshape(1, do))


def kernel(x, edge_index, W1, b1, Wmu, bmu, Wls, bls):
    src = edge_index[0].reshape(NW, EPW)
    dst = edge_index[1].reshape(NW, EPW)
    wids = jnp.arange(NW, dtype=jnp.int32)[:, None]
    padfill = jnp.broadcast_to(wids, (NW, PAD))
    srcp = jnp.concatenate([src, padfill], axis=1).reshape(NW, NCH, CB, B)
    dstp = jnp.concatenate([dst, N + padfill], axis=1).reshape(NW, NB, B)
    zrows = jnp.zeros((B, D), jnp.float32)

    deg_parts = _deg_sc(dstp)
    dinv, y1 = _prescale(deg_parts.reshape(NC, NP, 1), x)
    z1 = _agg_sc(y1, srcp, dstp, zrows)
    y2 = _mid(z1, y1, dinv, W1, b1)
    z2 = _agg_sc(y2, srcp, dstp, zrows)
    mu, logstd = _heads(z2, y2, dinv, Wmu, bmu, Wls, bls)
    return (mu, logstd)


# speedup vs baseline: 1.0471x; 1.0471x over previous
import functools

import jax
import jax.numpy as jnp
from jax import lax
from jax.experimental import pallas as pl
from jax.experimental.pallas import tpu as pltpu
from jax.experimental.pallas import tpu_sc as plsc

N = 10000
E = 320000
D = 128
NC = 2
NS = 16
NW = NC * NS
EPW = E // NW
B = 128
CB = 8
NCH = 10
NB = NCH * CB
PAD = NB * B - EPW
NP = 10240
RPT = NP // NS

_mesh = plsc.VectorSubcoreMesh(core_axis_name="c", subcore_axis_name="s")


@functools.partial(
    pl.kernel,
    out_type=jax.ShapeDtypeStruct((NC, NP), jnp.float32),
    mesh=_mesh,
    scratch_types=[
        pltpu.VMEM((NB, B), jnp.int32),
        pltpu.VMEM((B,), jnp.float32),
        pltpu.VMEM((RPT,), jnp.float32),
        pltpu.VMEM_SHARED((NP,), jnp.float32),
    ],
)
def _deg_sc(dst_hbm, deg_hbm, dstv, ones_v, buf_v, deg_sh):
    c = lax.axis_index("c")
    s = lax.axis_index("s")
    wid = s * NC + c
    for i in range(B // 16):
        ones_v[pl.ds(16 * i, 16)] = jnp.ones((16,), jnp.float32)
    for i in range(RPT // 16):
        buf_v[pl.ds(16 * i, 16)] = jnp.zeros((16,), jnp.float32)
    pltpu.sync_copy(buf_v, deg_sh.at[pl.ds(s * RPT, RPT)])
    pltpu.sync_copy(dst_hbm.at[wid], dstv)
    plsc.subcore_barrier()
    for j in range(NB):
        pltpu.sync_copy(ones_v, deg_sh.at[dstv.at[j]], add=True)
    plsc.subcore_barrier()
    pltpu.sync_copy(deg_sh.at[pl.ds(s * RPT, RPT)], buf_v)
    pltpu.sync_copy(buf_v, deg_hbm.at[c, pl.ds(s * RPT, RPT)])


@functools.partial(
    pl.kernel,
    out_type=jax.ShapeDtypeStruct((NC, NP, D), jnp.float32),
    mesh=_mesh,
    scratch_types=[
        pltpu.VMEM((NB, B), jnp.int32),
        pltpu.VMEM((CB, B), jnp.int32),
        pltpu.VMEM((CB, B), jnp.int32),
        pltpu.VMEM((B, D), jnp.float32),
        pltpu.VMEM((B, D), jnp.float32),
        pltpu.VMEM_SHARED((NP, D), jnp.float32),
        pltpu.SemaphoreType.DMA,
        pltpu.SemaphoreType.DMA,
        pltpu.SemaphoreType.DMA,
        pltpu.SemaphoreType.DMA,
        pltpu.SemaphoreType.DMA,
    ],
)
def _agg_sc(y_hbm, src_hbm, dst_hbm, zrows_hbm, z_hbm,
            dstv, s0, s1, r0, r1, z_sh, gsem0, gsem1, ssem0, ssem1, isem):
    c = lax.axis_index("c")
    s = lax.axis_index("s")
    wid = s * NC + c
    svs, rows = (s0, s1), (r0, r1)
    gsems, ssems = (gsem0, gsem1), (ssem0, ssem1)
    pltpu.sync_copy(zrows_hbm, r0)
    for k in range(RPT // B):
        pltpu.sync_copy(r0, z_sh.at[pl.ds(s * RPT + k * B, B)])
    pltpu.sync_copy(dst_hbm.at[wid], dstv)
    pltpu.sync_copy(src_hbm.at[wid, 0], s0)
    plsc.subcore_barrier()

    def gather(g, p):
        cc = g // CB
        return pltpu.async_copy(
            y_hbm.at[svs[cc % 2].at[g % CB]], rows[p], gsems[p])

    icp = [None]
    gcp = [None, None]
    gcp[0] = gather(0, 0)
    gcp[1] = gather(1, 1)
    for ch in range(NCH):
        cp = ch % 2
        if ch + 1 < NCH:
            icp[0] = pltpu.async_copy(src_hbm.at[wid, ch + 1], svs[1 - cp], isem)
        for j in range(CB):
            g = ch * CB + j
            p = g % 2
            gcp[p].wait()
            pltpu.sync_copy(rows[p], z_sh.at[dstv.at[g]], add=True)
            if g + 2 < NB:
                if j == CB - 2:
                    icp[0].wait()
                gcp[p] = gather(g + 2, p)
    plsc.subcore_barrier()
    pltpu.sync_copy(
        z_sh.at[pl.ds(s * RPT, RPT)], z_hbm.at[c, pl.ds(s * RPT, RPT)])


def _prescale_tc(dp_ref, x_ref, dinv_ref, y_ref):
    deg = dp_ref[0, :N] + dp_ref[1, :N] + 1.0
    dinv = lax.rsqrt(deg)
    dinv_ref[...] = dinv
    y_ref[...] = x_ref[...] * dinv


def _prescale(deg_parts, x):
    return pl.pallas_call(
        _prescale_tc,
        out_shape=[
            jax.ShapeDtypeStruct((N, 1), jnp.float32),
            jax.ShapeDtypeStruct((N, D), jnp.float32),
        ],
    )(deg_parts, x)


def _mid_tc(z_ref, y_ref, dinv_ref, w_ref, b_ref, y2_ref):
    dinv = dinv_ref[...]
    p = (z_ref[0, :N] + z_ref[1, :N] + y_ref[...]) * dinv
    h = jnp.maximum(
        jnp.dot(p, w_ref[...], preferred_element_type=jnp.float32) + b_ref[...],
        0.0,
    )
    y2_ref[...] = h * dinv


def _mid(z1, y1, dinv, W1, b1):
    return pl.pallas_call(
        _mid_tc,
        out_shape=jax.ShapeDtypeStruct((N, D), jnp.float32),
    )(z1, y1, dinv, W1, b1.reshape(1, D))


def _out_tc(z_ref, y_ref, dinv_ref, wm_ref, bm_ref, wl_ref, bl_ref,
            mu_ref, ls_ref):
    q = (z_ref[0, :N] + z_ref[1, :N] + y_ref[...]) * dinv_ref[...]
    mu_ref[...] = (
        jnp.dot(q, wm_ref[...], preferred_element_type=jnp.float32) + bm_ref[...]
    )
    ls_ref[...] = (
        jnp.dot(q, wl_ref[...], preferred_element_type=jnp.float32) + bl_ref[...]
    )


def _heads(z2, y2, dinv, Wmu, bmu, Wls, bls):
    do = Wmu.shape[1]
    return pl.pallas_call(
        _out_tc,
        out_shape=[
            jax.ShapeDtypeStruct((N, do), jnp.float32),
            jax.ShapeDtypeStruct((N, do), jnp.float32),
        ],
    )(z2, y2, dinv, Wmu, bmu.reshape(1, do), Wls, bls.reshape(1, do))


def kernel(x, edge_index, W1, b1, Wmu, bmu, Wls, bls):
    src = edge_index[0].reshape(NW, EPW)
    dst = edge_index[1].reshape(NW, EPW)
    wids = jnp.arange(NW, dtype=jnp.int32)[:, None]
    padfill = jnp.broadcast_to(wids, (NW, PAD))
    srcp = jnp.concatenate([src, padfill], axis=1).reshape(NW, NCH, CB, B)
    dstp = jnp.concatenate([dst, N + padfill], axis=1).reshape(NW, NB, B)
    zrows = jnp.zeros((B, D), jnp.float32)

    deg_parts = _deg_sc(dstp)
    dinv, y1 = _prescale(deg_parts.reshape(NC, NP, 1), x)
    z1 = _agg_sc(y1, srcp, dstp, zrows)
    y2 = _mid(z1, y1, dinv, W1, b1)
    z2 = _agg_sc(y2, srcp, dstp, zrows)
    mu, logstd = _heads(z2, y2, dinv, Wmu, bmu, Wls, bls)
    return (mu, logstd)


# speedup vs baseline: 1.0480x; 1.0009x over previous
import functools

import jax
import jax.numpy as jnp
from jax import lax
from jax.experimental import pallas as pl
from jax.experimental.pallas import tpu as pltpu
from jax.experimental.pallas import tpu_sc as plsc

N = 10000
E = 320000
D = 128
NC = 2
NS = 16
NW = NC * NS
EPW = E // NW
B = 128
CB = 8
NCH = 10
NB = NCH * CB
PAD = NB * B - EPW
NP = 10240
RPT = NP // NS

_mesh = plsc.VectorSubcoreMesh(core_axis_name="c", subcore_axis_name="s")


@functools.partial(
    pl.kernel,
    out_type=jax.ShapeDtypeStruct((NC, NP), jnp.float32),
    mesh=_mesh,
    scratch_types=[
        pltpu.VMEM((NB, B), jnp.int32),
        pltpu.VMEM((B,), jnp.float32),
        pltpu.VMEM((RPT,), jnp.float32),
        pltpu.VMEM_SHARED((NP,), jnp.float32),
    ],
)
def _deg_sc(dst_hbm, deg_hbm, dstv, ones_v, buf_v, deg_sh):
    c = lax.axis_index("c")
    s = lax.axis_index("s")
    wid = s * NC + c
    for i in range(B // 16):
        ones_v[pl.ds(16 * i, 16)] = jnp.ones((16,), jnp.float32)
    for i in range(RPT // 16):
        buf_v[pl.ds(16 * i, 16)] = jnp.zeros((16,), jnp.float32)
    pltpu.sync_copy(buf_v, deg_sh.at[pl.ds(s * RPT, RPT)])
    pltpu.sync_copy(dst_hbm.at[wid], dstv)
    plsc.subcore_barrier()
    for j in range(NB):
        pltpu.sync_copy(ones_v, deg_sh.at[dstv.at[j]], add=True)
    plsc.subcore_barrier()
    pltpu.sync_copy(deg_sh.at[pl.ds(s * RPT, RPT)], buf_v)
    pltpu.sync_copy(buf_v, deg_hbm.at[c, pl.ds(s * RPT, RPT)])


@functools.partial(
    pl.kernel,
    out_type=jax.ShapeDtypeStruct((NC, NP, D), jnp.float32),
    mesh=_mesh,
    scratch_types=[
        pltpu.VMEM((NB, B), jnp.int32),
        pltpu.VMEM((CB, B), jnp.int32),
        pltpu.VMEM((CB, B), jnp.int32),
        pltpu.VMEM((B, D), jnp.float32),
        pltpu.VMEM((B, D), jnp.float32),
        pltpu.VMEM_SHARED((NP, D), jnp.float32),
        pltpu.SemaphoreType.DMA,
        pltpu.SemaphoreType.DMA,
        pltpu.SemaphoreType.DMA,
    ],
)
def _agg_sc(y_hbm, src_hbm, dst_hbm, zrows_hbm, z_hbm,
            dstv, s0, s1, r0, r1, z_sh, gsem0, gsem1, isem):
    c = lax.axis_index("c")
    s = lax.axis_index("s")
    wid = s * NC + c
    svs, rows = (s0, s1), (r0, r1)
    gsems = (gsem0, gsem1)
    pltpu.sync_copy(zrows_hbm, r0)
    for k in range(RPT // B):
        pltpu.sync_copy(r0, z_sh.at[pl.ds(s * RPT + k * B, B)])
    pltpu.sync_copy(dst_hbm.at[wid], dstv)
    pltpu.sync_copy(src_hbm.at[wid, 0], s0)
    plsc.subcore_barrier()

    def gather(g, p):
        cc = g // CB
        return pltpu.async_copy(
            y_hbm.at[svs[cc % 2].at[g % CB]], rows[p], gsems[p])

    icp = [None]
    gcp = [None, None]
    gcp[0] = gather(0, 0)
    gcp[1] = gather(1, 1)
    for ch in range(NCH):
        cp = ch % 2
        if ch + 1 < NCH:
            icp[0] = pltpu.async_copy(src_hbm.at[wid, ch + 1], svs[1 - cp], isem)
        for j in range(CB):
            g = ch * CB + j
            p = g % 2
            gcp[p].wait()
            pltpu.sync_copy(rows[p], z_sh.at[dstv.at[g]], add=True)
            if g + 2 < NB:
                if j == CB - 2:
                    icp[0].wait()
                gcp[p] = gather(g + 2, p)
    plsc.subcore_barrier()
    pltpu.sync_copy(
        z_sh.at[pl.ds(s * RPT, RPT)], z_hbm.at[c, pl.ds(s * RPT, RPT)])


def _prescale_tc(dp_ref, x_ref, dinv_ref, y_ref):
    deg = dp_ref[0, :N] + dp_ref[1, :N] + 1.0
    dinv = lax.rsqrt(deg)
    dinv_ref[...] = dinv
    y_ref[...] = x_ref[...] * dinv


def _prescale(deg_parts, x):
    return pl.pallas_call(
        _prescale_tc,
        out_shape=[
            jax.ShapeDtypeStruct((N, 1), jnp.float32),
            jax.ShapeDtypeStruct((N, D), jnp.float32),
        ],
    )(deg_parts, x)


def _mid_tc(z_ref, y_ref, dinv_ref, w_ref, b_ref, y2_ref):
    dinv = dinv_ref[...]
    p = (z_ref[0, :N] + z_ref[1, :N] + y_ref[...]) * dinv
    h = jnp.maximum(
        jnp.dot(p, w_ref[...], preferred_element_type=jnp.float32) + b_ref[...],
        0.0,
    )
    y2_ref[...] = h * dinv


def _mid(z1, y1, dinv, W1, b1):
    return pl.pallas_call(
        _mid_tc,
        out_shape=jax.ShapeDtypeStruct((N, D), jnp.float32),
    )(z1, y1, dinv, W1, b1.reshape(1, D))


def _out_tc(z_ref, y_ref, dinv_ref, wm_ref, bm_ref, wl_ref, bl_ref,
            mu_ref, ls_ref):
    q = (z_ref[0, :N] + z_ref[1, :N] + y_ref[...]) * dinv_ref[...]
    mu_ref[...] = (
        jnp.dot(q, wm_ref[...], preferred_element_type=jnp.float32) + bm_ref[...]
    )
    ls_ref[...] = (
        jnp.dot(q, wl_ref[...], preferred_element_type=jnp.float32) + bl_ref[...]
    )


def _heads(z2, y2, dinv, Wmu, bmu, Wls, bls):
    do = Wmu.shape[1]
    return pl.pallas_call(
        _out_tc,
        out_shape=[
            jax.ShapeDtypeStruct((N, do), jnp.float32),
            jax.ShapeDtypeStruct((N, do), jnp.float32),
        ],
    )(z2, y2, dinv, Wmu, bmu.reshape(1, do), Wls, bls.reshape(1, do))


def kernel(x, edge_index, W1, b1, Wmu, bmu, Wls, bls):
    src = edge_index[0].reshape(NW, EPW)
    dst = edge_index[1].reshape(NW, EPW)
    wids = jnp.arange(NW, dtype=jnp.int32)[:, None]
    padfill = jnp.broadcast_to(wids, (NW, PAD))
    srcp = jnp.concatenate([src, padfill], axis=1).reshape(NW, NCH, CB, B)
    dstp = jnp.concatenate([dst, N + padfill], axis=1).reshape(NW, NB, B)
    zrows = jnp.zeros((B, D), jnp.float32)

    deg_parts = _deg_sc(dstp)
    dinv, y1 = _prescale(deg_parts.reshape(NC, NP, 1), x)
    z1 = _agg_sc(y1, srcp, dstp, zrows)
    y2 = _mid(z1, y1, dinv, W1, b1)
    z2 = _agg_sc(y2, srcp, dstp, zrows)
    mu, logstd = _heads(z2, y2, dinv, Wmu, bmu, Wls, bls)
    return (mu, logstd)
